# baseline (speedup 1.0000x reference)
import functools
import jax
import jax.numpy as jnp
from jax import lax
from jax.experimental import pallas as pl
from jax.experimental.pallas import tpu as pltpu
from jax.experimental.pallas import tpu_sc as plsc

_C = 768
_B = 4
_H = 224
_W = 224
_BH = 16
_NH = _H // _BH
_SPIN = 200_000


def _sc_spin_body(out_hbm, acc, sem):
    c = lax.axis_index("c")
    s = lax.axis_index("s")
    wid = s * 2 + c

    def body(i, v):
        return v * 1.0000001 + 1.0

    v = lax.fori_loop(0, _SPIN, body, jnp.zeros((16,), jnp.float32))
    acc[...] = v
    pltpu.sync_copy(acc, out_hbm.at[wid])


_sc_spin = functools.partial(
    pl.kernel,
    mesh=plsc.VectorSubcoreMesh(core_axis_name="c", subcore_axis_name="s"),
    out_type=jax.ShapeDtypeStruct((32, 16), jnp.float32),
    scratch_types=[
        pltpu.VMEM((16,), jnp.float32),
        pltpu.SemaphoreType.DMA,
    ],
)(_sc_spin_body)


def _copy_kernel(x_ref, out_ref):
    out_ref[...] = x_ref[...]


def _tc_copy(x):
    return pl.pallas_call(
        _copy_kernel,
        grid=(_B, _NH),
        in_specs=[pl.BlockSpec((1, _BH, _W, _C), lambda bi, hi: (bi, hi, 0, 0))],
        out_specs=pl.BlockSpec((1, _BH, _W, _C), lambda bi, hi: (bi, hi, 0, 0)),
        out_shape=jax.ShapeDtypeStruct((_B, _H, _W, _C), x.dtype),
    )(x)


def kernel(x, W, b):
    spin = _sc_spin()
    y = _tc_copy(x)
    return {"sc": spin, "tc": y}


# speedup vs baseline: 1.2148x; 1.2148x over previous
import jax
import jax.numpy as jnp
from jax.experimental import pallas as pl
from jax.experimental.pallas import tpu as pltpu

_C = 768
_K = 384
_B = 4
_H = 224
_W = 224
_BH = 8
_NB = _H // _BH
_RES = 8
_HEAD = _NB - _RES
_NS = 2 * _NB


def _fused_kernel(x_ref, w_ref, b_ref, out_ref, pool_ref, mask_ref, res_ref):
    bi = pl.program_id(0)
    p = pl.program_id(1)
    hi = pl.program_id(2)
    hx = hi // 2
    half = hi % 2

    @pl.when(p == 0)
    def _pool_phase():
        sub = x_ref[0, pl.ds(half * 4, 4), :, :]
        s = jnp.sum(sub, axis=(0, 1), keepdims=False).reshape(1, _C)

        @pl.when(hi == 0)
        def _init():
            pool_ref[...] = s

        @pl.when(hi != 0)
        def _acc():
            pool_ref[...] = pool_ref[...] + s

        @pl.when(hx >= _HEAD)
        def _keep():
            base = (hx - _HEAD) * _BH + half * 4

            def _copy_rows(r, _):
                res_ref[pl.ds(base + r * 2, 2), :, :] = x_ref[
                    0, pl.ds(half * 4 + r * 2, 2), :, :
                ]
                return 0

            jax.lax.fori_loop(0, 2, _copy_rows, 0)

        @pl.when(hi == _NS - 1)
        def _mask():
            row = pool_ref[...] * (1.0 / (_H * _W))
            scores = jax.nn.sigmoid(
                jnp.dot(row, w_ref[...], preferred_element_type=jnp.float32)
                + b_ref[...]
            )
            sc = scores.reshape(_C, 1)
            _CH = 128
            for c in range(_C // _CH):
                sch = scores[:, c * _CH:(c + 1) * _CH]
                idx_i = jax.lax.broadcasted_iota(jnp.int32, (_C, _CH), 0)
                idx_j = jax.lax.broadcasted_iota(jnp.int32, (_C, _CH), 1) + c * _CH
                greater = (sc > sch).astype(jnp.float32)
                eq_before = ((sc == sch) & (idx_i < idx_j)).astype(jnp.float32)
                rank = jnp.sum(greater + eq_before, axis=0, keepdims=True)
                mask_ref[:, pl.ds(c * _CH, _CH)] = (rank < _K).astype(jnp.float32)

    @pl.when((p == 1) & (hx < _HEAD))
    def _apply_stream():
        m = mask_ref[...].reshape(1, 1, _C)
        out_ref[0] = x_ref[0, pl.ds(half * 4, 4), :, :] * m

    @pl.when((p == 1) & (hx >= _HEAD))
    def _apply_resident():
        m = mask_ref[...].reshape(1, 1, _C)
        blk = res_ref[pl.ds((hx - _HEAD) * _BH + half * 4, 4), :, :]
        out_ref[0] = blk * m


def kernel(x, W, b):
    b2 = b.reshape(1, _C).astype(jnp.float32)

    def x_map(bi, p, hi):
        hx = hi // 2
        return (bi, jnp.where((p == 1) & (hx >= _HEAD), _HEAD - 1, hx), 0, 0)

    def out_map(bi, p, hi):
        return (bi, jnp.where(p == 0, 0, hi), 0, 0)

    out = pl.pallas_call(
        _fused_kernel,
        grid=(_B, 2, _NS),
        in_specs=[
            pl.BlockSpec((1, _BH, _W, _C), x_map),
            pl.BlockSpec((_C, _C), lambda bi, p, hi: (0, 0)),
            pl.BlockSpec((1, _C), lambda bi, p, hi: (0, 0)),
        ],
        out_specs=pl.BlockSpec((1, 4, _W, _C), out_map),
        out_shape=jax.ShapeDtypeStruct((_B, _H, _W, _C), x.dtype),
        compiler_params=pltpu.CompilerParams(
            vmem_limit_bytes=66_000_000,
            dimension_semantics=("parallel", "arbitrary", "arbitrary"),
        ),
        scratch_shapes=[
            pltpu.VMEM((1, _C), jnp.float32),
            pltpu.VMEM((1, _C), jnp.float32),
            pltpu.VMEM((_RES * _BH, _W, _C), jnp.float32),
        ],
    )(x, W, b2)
    return out


# speedup vs baseline: 1.7873x; 1.4713x over previous
import jax
import jax.numpy as jnp
from jax.experimental import pallas as pl
from jax.experimental.pallas import tpu as pltpu

_C = 768
_K = 384
_B = 4
_H = 224
_W = 224
_BH = 8
_NH = _H // _BH
_RES = 7
_HEAD = _NH - _RES


def _fused_kernel(x_ref, w_ref, b_ref, out_ref, pool_ref, mask_ref, res_ref):
    p = pl.program_id(1)
    bi = pl.program_id(0)
    hi = pl.program_id(2)

    @pl.when(p == 0)
    def _pool_phase():
        blk = x_ref[...]
        s = jnp.sum(blk[0], axis=(0, 1), keepdims=False).reshape(1, _C)

        @pl.when(hi == 0)
        def _init():
            pool_ref[...] = s

        @pl.when(hi != 0)
        def _acc():
            pool_ref[...] = pool_ref[...] + s

        @pl.when(hi >= _HEAD)
        def _keep():
            base = (hi - _HEAD) * _BH

            def _copy_rows(r, _):
                res_ref[pl.ds(base + r * 2, 2), :, :] = x_ref[0, pl.ds(r * 2, 2), :, :]
                return 0

            jax.lax.fori_loop(0, _BH // 2, _copy_rows, 0)

        @pl.when(hi == _NH - 1)
        def _mask():
            row = pool_ref[...] * (1.0 / (_H * _W))
            scores = jax.nn.sigmoid(
                jnp.dot(row, w_ref[...], preferred_element_type=jnp.float32)
                + b_ref[...]
            )
            sc = scores.reshape(_C, 1)
            _CH = 128
            for c in range(_C // _CH):
                sch = scores[:, c * _CH:(c + 1) * _CH]
                idx_i = jax.lax.broadcasted_iota(jnp.int32, (_C, _CH), 0)
                idx_j = jax.lax.broadcasted_iota(jnp.int32, (_C, _CH), 1) + c * _CH
                greater = (sc > sch).astype(jnp.float32)
                eq_before = ((sc == sch) & (idx_i < idx_j)).astype(jnp.float32)
                rank = jnp.sum(greater + eq_before, axis=0, keepdims=True)
                mask_ref[:, pl.ds(c * _CH, _CH)] = (rank < _K).astype(jnp.float32)

    @pl.when((p == 1) & (hi < _HEAD))
    def _apply_stream():
        m = mask_ref[...].reshape(1, 1, 1, _C)
        out_ref[...] = x_ref[...] * m

    @pl.when((p == 1) & (hi >= _HEAD))
    def _apply_resident():
        m = mask_ref[...].reshape(1, 1, _C)
        blk = res_ref[pl.ds((hi - _HEAD) * _BH, _BH), :, :]
        out_ref[0] = blk * m


def kernel(x, W, b):
    b2 = b.reshape(1, _C).astype(jnp.float32)

    def x_map(bi, p, hi):
        return (bi, jnp.where((p == 1) & (hi >= _HEAD), _HEAD - 1, hi), 0, 0)

    def out_map(bi, p, hi):
        return (bi, jnp.where(p == 0, 0, hi), 0, 0)

    out = pl.pallas_call(
        _fused_kernel,
        grid=(_B, 2, _NH),
        in_specs=[
            pl.BlockSpec((1, _BH, _W, _C), x_map),
            pl.BlockSpec((_C, _C), lambda bi, p, hi: (0, 0)),
            pl.BlockSpec((1, _C), lambda bi, p, hi: (0, 0)),
        ],
        out_specs=pl.BlockSpec((1, _BH, _W, _C), out_map),
        out_shape=jax.ShapeDtypeStruct((_B, _H, _W, _C), x.dtype),
        compiler_params=pltpu.CompilerParams(
            vmem_limit_bytes=66_000_000,
            dimension_semantics=("parallel", "arbitrary", "arbitrary"),
        ),
        scratch_shapes=[
            pltpu.VMEM((1, _C), jnp.float32),
            pltpu.VMEM((1, _C), jnp.float32),
            pltpu.VMEM((_RES * _BH, _W, _C), jnp.float32),
        ],
    )(x, W, b2)
    return out
